# Initial kernel scaffold; baseline (speedup 1.0000x reference)
import jax
import jax.numpy as jnp
from jax.experimental import pallas as pl


def kernel(pred_body, label, label_weight, sketch_gt):
    raise NotImplementedError("write your pallas kernel here")


# speedup vs baseline: 3.0900x; 3.0900x over previous
import jax
import jax.numpy as jnp
from jax.experimental import pallas as pl
from jax.experimental.pallas import tpu as pltpu

_C = 12
_SUB = 512
_LANE = 128
_T = _SUB * _LANE


def _body(pred_ref, lab_ref, lw_ref, sk_ref, num_ref, cnt_ref):
    @pl.when((pl.program_id(0) == 0) & (pl.program_id(1) == 0))
    def _init():
        num_ref[0, 0] = 0.0
        cnt_ref[0, 0] = 0.0

    x = pred_ref[0, :, 0]
    m = jnp.max(x, axis=0)
    s = jnp.sum(jnp.exp(x - m[None]), axis=0)
    lse = jnp.log(s) + m

    lab = lab_ref[0, 0]
    cls = jax.lax.broadcasted_iota(jnp.int32, (_C, _SUB, _LANE), 0)
    pick = jnp.sum(jnp.where(cls == lab[None], x, 0.0), axis=0)

    mask = (sk_ref[0, 0] == 0) & (lw_ref[0, 0] != 0)
    ce = jnp.where(mask & (lab != 255), lse - pick, 0.0)
    num_ref[0, 0] += jnp.sum(ce)
    cnt_ref[0, 0] += jnp.sum(mask.astype(jnp.float32))


def kernel(pred_body, label, label_weight, sketch_gt):
    B, C = pred_body.shape[:2]
    S = pred_body.shape[2] * pred_body.shape[3] * pred_body.shape[4]
    nchunks = S // _T

    pred = pred_body.reshape(B, C, nchunks, _SUB, _LANE)
    lab = label.reshape(B, nchunks, _SUB, _LANE).astype(jnp.int32)
    lw = label_weight.reshape(B, nchunks, _SUB, _LANE)
    sk = sketch_gt.reshape(B, nchunks, _SUB, _LANE)

    vox_spec = pl.BlockSpec((1, 1, _SUB, _LANE), lambda b, i: (b, i, 0, 0))
    out_spec = pl.BlockSpec(memory_space=pltpu.SMEM)
    num, cnt = pl.pallas_call(
        _body,
        grid=(B, nchunks),
        in_specs=[
            pl.BlockSpec((1, C, 1, _SUB, _LANE), lambda b, i: (b, 0, i, 0, 0)),
            vox_spec,
            vox_spec,
            vox_spec,
        ],
        out_specs=[out_spec, out_spec],
        out_shape=[
            jax.ShapeDtypeStruct((1, 1), jnp.float32),
            jax.ShapeDtypeStruct((1, 1), jnp.float32),
        ],
    )(pred, lab, lw, sk)
    return num[0, 0] / cnt[0, 0]


# speedup vs baseline: 10.3302x; 3.3431x over previous
import jax
import jax.numpy as jnp
from jax.experimental import pallas as pl
from jax.experimental.pallas import tpu as pltpu

_C = 12


def _body(pred_ref, lab_ref, lw_ref, sk_ref, num_ref, cnt_ref):
    @pl.when(pl.program_id(0) == 0)
    def _init():
        num_ref[0, 0] = 0.0
        cnt_ref[0, 0] = 0.0

    B, _, _, H, W = pred_ref.shape
    x = pred_ref[:, :, 0]
    m = jnp.max(x, axis=1)
    s = jnp.sum(jnp.exp(x - m[:, None]), axis=1)
    lse = jnp.log(s) + m

    lab = lab_ref[:, 0]
    cls = jax.lax.broadcasted_iota(jnp.int32, (B, _C, H, W), 1)
    pick = jnp.sum(jnp.where(cls == lab[:, None], x, 0.0), axis=1)

    lw = lw_ref[...].reshape(B, H, W)
    mask = (sk_ref[:, 0] == 0) & (lw != 0)
    ce = jnp.where(mask & (lab != 255), lse - pick, 0.0)
    num_ref[0, 0] += jnp.sum(ce)
    cnt_ref[0, 0] += jnp.sum(mask.astype(jnp.float32))


def kernel(pred_body, label, label_weight, sketch_gt):
    B, C, D, H, W = pred_body.shape
    vox_spec = pl.BlockSpec((B, 1, H, W), lambda d: (0, d, 0, 0))
    out_spec = pl.BlockSpec(memory_space=pltpu.SMEM)
    num, cnt = pl.pallas_call(
        _body,
        grid=(D,),
        in_specs=[
            pl.BlockSpec((B, C, 1, H, W), lambda d: (0, 0, d, 0, 0)),
            vox_spec,
            pl.BlockSpec((B, H * W), lambda d: (0, d)),
            vox_spec,
        ],
        out_specs=[out_spec, out_spec],
        out_shape=[
            jax.ShapeDtypeStruct((1, 1), jnp.float32),
            jax.ShapeDtypeStruct((1, 1), jnp.float32),
        ],
    )(pred_body, label.astype(jnp.int32), label_weight, sketch_gt)
    return num[0, 0] / cnt[0, 0]
